# baseline (speedup 1.0000x reference)
import jax
import jax.numpy as jnp
from jax.experimental import pallas as pl
from jax.experimental.pallas import tpu as pltpu

N = 19
E = 342
D = 512


def _pair_logits(xl, xr, a2d):
    f32 = jnp.float32
    u = jax.lax.dot_general(a2d, xl, (((1,), (1,)), ((), ())),
                            preferred_element_type=f32)
    v = jax.lax.dot_general(a2d, xr, (((1,), (1,)), ((), ())),
                            preferred_element_type=f32)
    rows = []
    for s in range(N):
        az = jnp.abs(xl[s:s + 1, :] + xr)
        w = jax.lax.dot_general(a2d, az, (((1,), (1,)), ((), ())),
                                preferred_element_type=f32)
        rows.append(0.6 * (u[0:1, s:s + 1] + v) + 0.4 * w)
    return jnp.concatenate(rows, axis=0)


def _fused(x_ref, ei_ref, mmse_ref, wl1_ref, wr1_ref, a1_ref, b1_ref,
           wl2_ref, wr2_ref, a2_ref, b2_ref, wm_ref, bm_ref,
           comp_ref, alpha_ref,
           w1s, w2s, w3s, w4s, s1, s2, s3, s4):
    f32 = jnp.float32
    cp1 = pltpu.make_async_copy(wl1_ref, w1s, s1)
    cp2 = pltpu.make_async_copy(wr1_ref, w2s, s2)
    cp3 = pltpu.make_async_copy(wl2_ref, w3s, s3)
    cp4 = pltpu.make_async_copy(wr2_ref, w4s, s4)
    cp1.start()
    cp2.start()
    cp3.start()
    cp4.start()

    src = ei_ref[0:1, :]
    dst = ei_ref[1:2, :]
    iota_ne = jax.lax.broadcasted_iota(jnp.int32, (N, E), 0)
    s_oh = (iota_ne == src).astype(f32)
    d_oh = (iota_ne == dst).astype(f32)
    c2 = jax.lax.dot_general(s_oh, d_oh, (((1,), (1,)), ((), ())),
                             preferred_element_type=f32)
    has = c2 > 0.0

    def gatv2(h, wl, wr, a, b):
        xl = jnp.dot(h, wl, preferred_element_type=f32)
        xr = jnp.dot(h, wr, preferred_element_type=f32)
        e2 = _pair_logits(xl, xr, a)
        m = jnp.max(jnp.where(has, e2, -1e30), axis=0, keepdims=True)
        ex = jnp.where(has, jnp.exp(e2 - m), 0.0)
        ssum = jnp.sum(c2 * ex, axis=0, keepdims=True)
        alpha = ex / (ssum + 1e-16)
        wmat = c2 * alpha
        out = jax.lax.dot_general(wmat, xl, (((0,), (0,)), ((), ())),
                                  preferred_element_type=f32)
        return out + b, alpha

    cp1.wait()
    cp2.wait()
    h1, alpha1 = gatv2(x_ref[...], w1s[...], w2s[...],
                       a1_ref[...].reshape(1, D), b1_ref[...].reshape(1, D))
    u = jax.lax.dot_general(alpha1, d_oh, (((1,), (0,)), ((), ())),
                            preferred_element_type=f32)
    alpha_ref[...] = jnp.sum(s_oh * u, axis=0, keepdims=True)

    cp3.wait()
    cp4.wait()
    h2, _ = gatv2(h1, w3s[...], w4s[...],
                  a2_ref[...].reshape(1, D), b2_ref[...].reshape(1, D))
    gf = h2 + mmse_ref[...] * wm_ref[...] + bm_ref[...].reshape(1, D)
    dec = jax.lax.dot_general(gf, gf, (((1,), (1,)), ((), ())),
                              preferred_element_type=f32)
    comp_ref[...] = jax.nn.sigmoid(dec)


def kernel(x, edge_index, mmse, Wl1, Wr1, a1, b1, Wl2, Wr2, a2, b2, Wm, bm,
           W11, b11, W12, b12, W21, b21, W22, b22, Wp, bp):
    f32 = jnp.float32
    T = x.shape[1]
    vm = pl.BlockSpec(memory_space=pl.ANY)
    compressed, alpha_2d = pl.pallas_call(
        _fused,
        in_specs=[pl.BlockSpec((19, T), lambda: (0, 0)),
                  pl.BlockSpec((2, E), lambda: (0, 0)),
                  pl.BlockSpec((1, 1), lambda: (0, 0)),
                  vm, vm,
                  pl.BlockSpec((D,), lambda: (0,)),
                  pl.BlockSpec((D,), lambda: (0,)),
                  vm, vm,
                  pl.BlockSpec((D,), lambda: (0,)),
                  pl.BlockSpec((D,), lambda: (0,)),
                  pl.BlockSpec((1, D), lambda: (0, 0)),
                  pl.BlockSpec((D,), lambda: (0,))],
        out_shape=[
            jax.ShapeDtypeStruct((N, N), f32),
            jax.ShapeDtypeStruct((1, E), f32),
        ],
        scratch_shapes=[
            pltpu.VMEM((T, D), f32),
            pltpu.VMEM((T, D), f32),
            pltpu.VMEM((D, D), f32),
            pltpu.VMEM((D, D), f32),
            pltpu.SemaphoreType.DMA,
            pltpu.SemaphoreType.DMA,
            pltpu.SemaphoreType.DMA,
            pltpu.SemaphoreType.DMA,
        ],
    )(x, edge_index, mmse.reshape(1, 1),
      Wl1, Wr1, a1, b1, Wl2, Wr2, a2, b2, Wm, bm)
    return compressed, alpha_2d.reshape(E)


# speedup vs baseline: 1.2579x; 1.2579x over previous
import jax
import jax.numpy as jnp
from jax.experimental import pallas as pl

N = 19
E = 342
D = 512
_HI = jax.lax.Precision.DEFAULT


def _pair_logits(xl, xr, a2d):
    f32 = jnp.float32
    u = jax.lax.dot_general(a2d, xl, (((1,), (1,)), ((), ())),
                            precision=_HI, preferred_element_type=f32)
    v = jax.lax.dot_general(a2d, xr, (((1,), (1,)), ((), ())),
                            precision=_HI, preferred_element_type=f32)
    rows = []
    for s in range(N):
        az = jnp.abs(xl[s:s + 1, :] + xr)
        w = jax.lax.dot_general(a2d, az, (((1,), (1,)), ((), ())),
                                precision=_HI, preferred_element_type=f32)
        rows.append(0.6 * (u[0:1, s:s + 1] + v) + 0.4 * w)
    return jnp.concatenate(rows, axis=0)


def _fused(x_ref, ei_ref, mmse_ref, wl1_ref, wr1_ref, a1_ref, b1_ref,
           wl2_ref, wr2_ref, a2_ref, b2_ref, wm_ref, bm_ref,
           comp_ref, alpha_ref):
    f32 = jnp.float32
    src = ei_ref[0:1, :]
    dst = ei_ref[1:2, :]
    iota_ne = jax.lax.broadcasted_iota(jnp.int32, (N, E), 0)
    s_oh = (iota_ne == src).astype(f32)
    d_oh = (iota_ne == dst).astype(f32)
    c2 = jax.lax.dot_general(s_oh, d_oh, (((1,), (1,)), ((), ())),
                             preferred_element_type=f32)
    has = c2 > 0.0

    def gatv2(h, wl, wr, a, b):
        xl = jnp.dot(h, wl, precision=_HI, preferred_element_type=f32)
        xr = jnp.dot(h, wr, precision=_HI, preferred_element_type=f32)
        e2 = _pair_logits(xl, xr, a)
        m = jnp.max(jnp.where(has, e2, -1e30), axis=0, keepdims=True)
        ex = jnp.where(has, jnp.exp(e2 - m), 0.0)
        ssum = jnp.sum(c2 * ex, axis=0, keepdims=True)
        alpha = ex / (ssum + 1e-16)
        wmat = c2 * alpha
        out = jax.lax.dot_general(wmat, xl, (((0,), (0,)), ((), ())),
                                  precision=_HI, preferred_element_type=f32)
        return out + b, alpha

    h1, alpha1 = gatv2(x_ref[...], wl1_ref[...], wr1_ref[...],
                       a1_ref[...].reshape(1, D), b1_ref[...].reshape(1, D))
    h2, _ = gatv2(h1, wl2_ref[...], wr2_ref[...],
                  a2_ref[...].reshape(1, D), b2_ref[...].reshape(1, D))
    gf = h2 + mmse_ref[...].reshape(1, 1) * wm_ref[...] + bm_ref[...].reshape(1, D)
    dec = jax.lax.dot_general(gf, gf, (((1,), (1,)), ((), ())),
                              precision=_HI, preferred_element_type=f32)
    comp_ref[...] = jax.nn.sigmoid(dec)
    u = jax.lax.dot_general(alpha1, d_oh, (((1,), (0,)), ((), ())),
                            precision=_HI, preferred_element_type=f32)
    alpha_ref[...] = jnp.sum(s_oh * u, axis=0)


def kernel(x, edge_index, mmse, Wl1, Wr1, a1, b1, Wl2, Wr2, a2, b2, Wm, bm,
           W11, b11, W12, b12, W21, b21, W22, b22, Wp, bp):
    compressed, alpha_1d = pl.pallas_call(
        _fused,
        out_shape=[
            jax.ShapeDtypeStruct((N, N), jnp.float32),
            jax.ShapeDtypeStruct((E,), jnp.float32),
        ],
    )(x, edge_index, mmse,
      Wl1, Wr1, a1, b1, Wl2, Wr2, a2, b2, Wm, bm)
    return compressed, alpha_1d
